# baseline (speedup 1.0000x reference)
import functools

import jax
import jax.numpy as jnp
from jax.experimental import pallas as pl
from jax.experimental.pallas import tpu as pltpu

_N_TOKENS = 32768
_D_MODEL = 768
_N_EXPERTS = 64
_BLOCK_T = 4096


def _router_kernel(x_ref, wgt_ref, gates_ref, idx_ref, aux_ref,
                   gsum_ref, psum_ref, *, num_blocks, n_tokens):
    i = pl.program_id(0)
    logits = jax.lax.dot_general(
        x_ref[...], wgt_ref[...], (((1,), (1,)), ((), ())),
        preferred_element_type=jnp.float32)

    m1 = jnp.max(logits, axis=-1, keepdims=True)
    a1 = jnp.argmax(logits, axis=-1)
    eidx = jax.lax.broadcasted_iota(jnp.int32, logits.shape, 1)
    hot1 = eidx == a1[:, None]
    masked = jnp.where(hot1, -jnp.inf, logits)
    m2 = jnp.max(masked, axis=-1, keepdims=True)
    a2 = jnp.argmax(masked, axis=-1)
    hot2 = eidx == a2[:, None]

    t = jnp.exp(m2 - m1)
    s = 1.0 + t
    w1 = 1.0 / s
    w2 = t / s
    gates = jnp.where(hot1, w1, 0.0) + jnp.where(hot2, w2, 0.0)
    gates_ref[...] = gates
    pair = jax.lax.broadcasted_iota(jnp.int32, (a1.shape[0], 2), 1)
    idx_ref[...] = jnp.where(pair == 0, a1[:, None], a2[:, None])

    p = jnp.exp(logits - m1)
    r = 1.0 / jnp.sum(p, axis=-1, keepdims=True)

    @pl.when(i == 0)
    def _init():
        gsum_ref[...] = jnp.zeros_like(gsum_ref)
        psum_ref[...] = jnp.zeros_like(psum_ref)

    ones_t = jnp.ones((gates.shape[0], 1), jnp.float32)
    gsum_ref[...] += jax.lax.dot_general(
        ones_t, gates, (((0,), (0,)), ((), ())),
        preferred_element_type=jnp.float32)
    psum_ref[...] += jax.lax.dot_general(
        r, p, (((0,), (0,)), ((), ())),
        preferred_element_type=jnp.float32)

    @pl.when(i == num_blocks - 1)
    def _finish():
        scale = jnp.float32(_N_EXPERTS) / (jnp.float32(n_tokens) ** 2)
        aux_ref[...] = jnp.sum(
            gsum_ref[...] * psum_ref[...], keepdims=True) * scale


def kernel(x, w_gate, w_noise):
    del w_noise
    n, d = x.shape
    e = w_gate.shape[0]
    t = _BLOCK_T
    num_blocks = n // t

    gates, idx, aux = pl.pallas_call(
        functools.partial(_router_kernel, num_blocks=num_blocks,
                          n_tokens=n),
        grid=(num_blocks,),
        in_specs=[
            pl.BlockSpec((t, d), lambda i: (i, 0)),
            pl.BlockSpec((e, d), lambda i: (0, 0)),
        ],
        out_specs=[
            pl.BlockSpec((t, e), lambda i: (i, 0)),
            pl.BlockSpec((t, 2), lambda i: (i, 0)),
            pl.BlockSpec((1, 1), lambda i: (0, 0)),
        ],
        out_shape=[
            jax.ShapeDtypeStruct((n, e), jnp.float32),
            jax.ShapeDtypeStruct((n, 2), jnp.int32),
            jax.ShapeDtypeStruct((1, 1), jnp.float32),
        ],
        scratch_shapes=[
            pltpu.VMEM((1, e), jnp.float32),
            pltpu.VMEM((1, e), jnp.float32),
        ],
    )(x, w_gate)
    return gates, idx, aux[0, 0]


# speedup vs baseline: 1.0095x; 1.0095x over previous
import functools

import jax
import jax.numpy as jnp
from jax.experimental import pallas as pl
from jax.experimental.pallas import tpu as pltpu

_N_TOKENS = 32768
_D_MODEL = 768
_N_EXPERTS = 64
_BLOCK_T = 4096


def _router_kernel(x_ref, wgt_ref, gates_ref, idx_ref, aux_ref,
                   gsum_ref, psum_ref, *, num_blocks, n_tokens):
    i = pl.program_id(0)
    logits = jax.lax.dot_general(
        x_ref[...], wgt_ref[...], (((1,), (1,)), ((), ())),
        preferred_element_type=jnp.float32)

    m1 = jnp.max(logits, axis=-1, keepdims=True)
    a1 = jnp.argmax(logits, axis=-1)
    eidx = jax.lax.broadcasted_iota(jnp.int32, logits.shape, 1)
    hot1 = eidx == a1[:, None]
    masked = jnp.where(hot1, -jnp.inf, logits)
    m2 = jnp.max(masked, axis=-1, keepdims=True)
    a2 = jnp.argmax(masked, axis=-1)
    hot2 = eidx == a2[:, None]

    t = jnp.exp(m2 - m1)
    s = 1.0 + t
    w1 = 1.0 / s
    w2 = t / s
    gates = jnp.where(hot1, w1, 0.0) + jnp.where(hot2, w2, 0.0)
    gates_ref[...] = gates
    pair = jax.lax.broadcasted_iota(jnp.int32, (a1.shape[0], 2), 1)
    idx_ref[...] = jnp.where(pair == 0, a1[:, None], a2[:, None])

    p = jnp.exp(logits - m1)
    probs = p / jnp.sum(p, axis=-1, keepdims=True)

    @pl.when(i == 0)
    def _init():
        gsum_ref[...] = jnp.zeros_like(gsum_ref)
        psum_ref[...] = jnp.zeros_like(psum_ref)

    gsum_ref[...] += jnp.sum(gates, axis=0, keepdims=True)
    psum_ref[...] += jnp.sum(probs, axis=0, keepdims=True)

    @pl.when(i == num_blocks - 1)
    def _finish():
        scale = jnp.float32(_N_EXPERTS) / (jnp.float32(n_tokens) ** 2)
        aux_ref[...] = jnp.sum(
            gsum_ref[...] * psum_ref[...], keepdims=True) * scale


def kernel(x, w_gate, w_noise):
    del w_noise
    n, d = x.shape
    e = w_gate.shape[0]
    t = _BLOCK_T
    num_blocks = n // t

    gates, idx, aux = pl.pallas_call(
        functools.partial(_router_kernel, num_blocks=num_blocks,
                          n_tokens=n),
        grid=(num_blocks,),
        in_specs=[
            pl.BlockSpec((t, d), lambda i: (i, 0)),
            pl.BlockSpec((e, d), lambda i: (0, 0)),
        ],
        out_specs=[
            pl.BlockSpec((t, e), lambda i: (i, 0)),
            pl.BlockSpec((t, 2), lambda i: (i, 0)),
            pl.BlockSpec((1, 1), lambda i: (0, 0)),
        ],
        out_shape=[
            jax.ShapeDtypeStruct((n, e), jnp.float32),
            jax.ShapeDtypeStruct((n, 2), jnp.int32),
            jax.ShapeDtypeStruct((1, 1), jnp.float32),
        ],
        scratch_shapes=[
            pltpu.VMEM((1, e), jnp.float32),
            pltpu.VMEM((1, e), jnp.float32),
        ],
    )(x, w_gate)
    return gates, idx, aux[0, 0]
